# baseline (speedup 1.0000x reference)
import functools

import jax
import jax.numpy as jnp
from jax import lax
from jax.experimental import pallas as pl
from jax.experimental.pallas import tpu as pltpu
from jax.experimental.pallas import tpu_sc as plsc

VOCAB = 1000000
EMB = 64
HID = 64
B = 4096
T = 200


_NC, _NS = 2, 16
_NW = _NC * _NS
_N_ROWS = B * T
_PER_W = _N_ROWS // _NW
_CH = 128
_NCHUNK = _PER_W // _CH


def _sc_gather(table, idx_2d):
    mesh = plsc.VectorSubcoreMesh(core_axis_name="c", subcore_axis_name="s")

    @functools.partial(
        pl.kernel,
        mesh=mesh,
        compiler_params=pltpu.CompilerParams(use_tc_tiling_on_sc=False),
        out_type=jax.ShapeDtypeStruct((_N_ROWS, EMB), jnp.float32),
        scratch_types=[
            pltpu.VMEM((_NCHUNK, _CH), jnp.int32),
            pltpu.VMEM((2, _CH, EMB), jnp.float32),
            pltpu.SemaphoreType.DMA,
            pltpu.SemaphoreType.DMA,
        ],
    )
    def k(table_hbm, idx_hbm, out_hbm, idx_v, rows_v, sem0, sem1):
        wid = lax.axis_index("s") * _NC + lax.axis_index("c")
        base = wid * _PER_W
        pltpu.sync_copy(idx_hbm.at[pl.ds(wid * _NCHUNK, _NCHUNK)], idx_v)
        sems = (sem0, sem1)

        def start(j, slot):
            pltpu.async_copy(table_hbm.at[idx_v.at[j]], rows_v.at[slot],
                             sems[slot])

        start(0, 0)
        start(1, 1)

        def pair(i, carry):
            for b in range(2):
                j = 2 * i + b
                pltpu.make_async_copy(table_hbm.at[idx_v.at[j]],
                                      rows_v.at[b], sems[b]).wait()
                pltpu.sync_copy(rows_v.at[b],
                                out_hbm.at[pl.ds(base + j * _CH, _CH)])

                @pl.when(j + 2 < _NCHUNK)
                def _start_next(b=b, j=j):
                    start(j + 2, b)
            return carry

        lax.fori_loop(0, _NCHUNK // 2, pair, 0)

    return k(table, idx_2d)


def _gru_math(x_t, h, wih, whh, bih, bhh):
    gi = lax.dot_general(x_t.astype(jnp.bfloat16), wih,
                         (((1,), (1,)), ((), ())),
                         preferred_element_type=jnp.float32) + bih
    gh = lax.dot_general(h.astype(jnp.bfloat16), whh,
                         (((1,), (1,)), ((), ())),
                         preferred_element_type=jnp.float32) + bhh
    r = jax.nn.sigmoid(gi[:, :HID] + gh[:, :HID])
    z = jax.nn.sigmoid(gi[:, HID:2 * HID] + gh[:, HID:2 * HID])
    n = jnp.tanh(gi[:, 2 * HID:] + r * gh[:, 2 * HID:])
    return (1.0 - z) * n + z * h


def _fwd_body(emb_ref, wih_ref, whh_ref, bih_ref, bhh_ref, out_ref, h_ref):
    t = pl.program_id(0)

    @pl.when(t == 0)
    def _():
        h_ref[...] = jnp.zeros((B, HID), jnp.float32)

    h_new = _gru_math(emb_ref[0], h_ref[...], wih_ref[...], whh_ref[...],
                      bih_ref[...], bhh_ref[...])
    h_ref[...] = h_new
    out_ref[0] = h_new


def _bwd_body(emb_ref, fwd_ref, wih_ref, whh_ref, bih_ref, bhh_ref,
              out_ref, h_ref):
    t = pl.program_id(0)

    @pl.when(t == 0)
    def _():
        h_ref[...] = jnp.zeros((B, HID), jnp.float32)

    h_new = _gru_math(emb_ref[0], h_ref[...], wih_ref[...], whh_ref[...],
                      bih_ref[...], bhh_ref[...])
    h_ref[...] = h_new
    j = (T - 1 - t) % 8
    out_ref[:, j, :] = jnp.concatenate([fwd_ref[0], h_new], axis=-1)


def _w_specs():
    return [
        pl.BlockSpec((3 * HID, EMB), lambda t: (0, 0)),
        pl.BlockSpec((3 * HID, HID), lambda t: (0, 0)),
        pl.BlockSpec((1, 3 * HID), lambda t: (0, 0)),
        pl.BlockSpec((1, 3 * HID), lambda t: (0, 0)),
    ]


def _prep_w(W_ih, W_hh, b_ih, b_hh):
    return (W_ih.astype(jnp.bfloat16), W_hh.astype(jnp.bfloat16),
            b_ih[None, :], b_hh[None, :])


def _gru_forward(emb_tbe, wih, whh, bih, bhh):
    return pl.pallas_call(
        _fwd_body,
        grid=(T,),
        in_specs=[pl.BlockSpec((1, B, EMB), lambda t: (t, 0, 0))] + _w_specs(),
        out_specs=pl.BlockSpec((1, B, HID), lambda t: (t, 0, 0)),
        out_shape=jax.ShapeDtypeStruct((T, B, HID), jnp.float32),
        scratch_shapes=[pltpu.VMEM((B, HID), jnp.float32)],
    )(emb_tbe, wih, whh, bih, bhh)


def _gru_backward_combine(emb_tbe, out_f, wih, whh, bih, bhh):
    rev = lambda t: (T - 1 - t, 0, 0)
    return pl.pallas_call(
        _bwd_body,
        grid=(T,),
        in_specs=[pl.BlockSpec((1, B, EMB), rev),
                  pl.BlockSpec((1, B, HID), rev)] + _w_specs(),
        out_specs=pl.BlockSpec((B, 8, 2 * HID),
                               lambda t: (0, (T - 1 - t) // 8, 0)),
        out_shape=jax.ShapeDtypeStruct((B, T, 2 * HID), jnp.float32),
        scratch_shapes=[pltpu.VMEM((B, HID), jnp.float32)],
    )(emb_tbe, out_f, wih, whh, bih, bhh)


def kernel(x, table, W_ih_f, W_hh_f, b_ih_f, b_hh_f,
           W_ih_b, W_hh_b, b_ih_b, b_hh_b):
    idx_2d = x.T.reshape(_NW * _NCHUNK, _CH).astype(jnp.int32)
    emb = _sc_gather(table, idx_2d).reshape(T, B, EMB)
    out_f = _gru_forward(emb, *_prep_w(W_ih_f, W_hh_f, b_ih_f, b_hh_f))
    return _gru_backward_combine(emb, out_f,
                                 *_prep_w(W_ih_b, W_hh_b, b_ih_b, b_hh_b))


# speedup vs baseline: 1.0173x; 1.0173x over previous
import functools

import jax
import jax.numpy as jnp
from jax import lax
from jax.experimental import pallas as pl
from jax.experimental.pallas import tpu as pltpu
from jax.experimental.pallas import tpu_sc as plsc

VOCAB = 1000000
EMB = 64
HID = 64
B = 4096
T = 200


_NC, _NS = 2, 16
_NW = _NC * _NS
_N_ROWS = B * T
_PER_W = _N_ROWS // _NW
_CH = 128
_NCHUNK = _PER_W // _CH


def _sc_gather(table, idx_2d):
    mesh = plsc.VectorSubcoreMesh(core_axis_name="c", subcore_axis_name="s")

    @functools.partial(
        pl.kernel,
        mesh=mesh,
        compiler_params=pltpu.CompilerParams(use_tc_tiling_on_sc=False),
        out_type=jax.ShapeDtypeStruct((_N_ROWS, EMB), jnp.float32),
        scratch_types=[
            pltpu.VMEM((_NCHUNK, _CH), jnp.int32),
            pltpu.VMEM((2, _CH, EMB), jnp.float32),
            pltpu.SemaphoreType.DMA,
            pltpu.SemaphoreType.DMA,
        ],
    )
    def k(table_hbm, idx_hbm, out_hbm, idx_v, rows_v, sem0, sem1):
        wid = lax.axis_index("s") * _NC + lax.axis_index("c")
        base = wid * _PER_W
        pltpu.sync_copy(idx_hbm.at[pl.ds(wid * _NCHUNK, _NCHUNK)], idx_v)
        sems = (sem0, sem1)

        def start(j, slot):
            pltpu.async_copy(table_hbm.at[idx_v.at[j]], rows_v.at[slot],
                             sems[slot])

        start(0, 0)
        start(1, 1)

        def pair(i, carry):
            for b in range(2):
                j = 2 * i + b
                pltpu.make_async_copy(table_hbm.at[idx_v.at[j]],
                                      rows_v.at[b], sems[b]).wait()
                pltpu.sync_copy(rows_v.at[b],
                                out_hbm.at[pl.ds(base + j * _CH, _CH)])

                @pl.when(j + 2 < _NCHUNK)
                def _start_next(b=b, j=j):
                    start(j + 2, b)
            return carry

        lax.fori_loop(0, _NCHUNK // 2, pair, 0)

    return k(table, idx_2d)


def _gru_math(x_t, h, wih, whh, bih, bhh):
    gi = lax.dot_general(x_t, wih, (((1,), (1,)), ((), ())),
                         preferred_element_type=jnp.float32) + bih
    gh = lax.dot_general(h, whh, (((1,), (1,)), ((), ())),
                         preferred_element_type=jnp.float32) + bhh
    r = jax.nn.sigmoid(gi[:, :HID] + gh[:, :HID])
    z = jax.nn.sigmoid(gi[:, HID:2 * HID] + gh[:, HID:2 * HID])
    n = jnp.tanh(gi[:, 2 * HID:] + r * gh[:, 2 * HID:])
    return (1.0 - z) * n + z * h


def _fwd_body(emb_ref, wih_ref, whh_ref, bih_ref, bhh_ref, out_ref, h_ref):
    t = pl.program_id(0)

    @pl.when(t == 0)
    def _():
        h_ref[...] = jnp.zeros((B, HID), jnp.float32)

    h_new = _gru_math(emb_ref[0], h_ref[...], wih_ref[...], whh_ref[...],
                      bih_ref[...], bhh_ref[...])
    h_ref[...] = h_new
    out_ref[0] = h_new


def _bwd_body(emb_ref, fwd_ref, wih_ref, whh_ref, bih_ref, bhh_ref,
              out_ref, h_ref):
    t = pl.program_id(0)

    @pl.when(t == 0)
    def _():
        h_ref[...] = jnp.zeros((B, HID), jnp.float32)

    h_new = _gru_math(emb_ref[0], h_ref[...], wih_ref[...], whh_ref[...],
                      bih_ref[...], bhh_ref[...])
    h_ref[...] = h_new
    j = (T - 1 - t) % 8
    out_ref[:, j, :] = jnp.concatenate([fwd_ref[0], h_new], axis=-1)


def _w_specs():
    return [
        pl.BlockSpec((3 * HID, EMB), lambda t: (0, 0)),
        pl.BlockSpec((3 * HID, HID), lambda t: (0, 0)),
        pl.BlockSpec((1, 3 * HID), lambda t: (0, 0)),
        pl.BlockSpec((1, 3 * HID), lambda t: (0, 0)),
    ]


def _prep_w(W_ih, W_hh, b_ih, b_hh):
    return (W_ih, W_hh, b_ih[None, :], b_hh[None, :])


def _gru_forward(emb_tbe, wih, whh, bih, bhh):
    return pl.pallas_call(
        _fwd_body,
        grid=(T,),
        in_specs=[pl.BlockSpec((1, B, EMB), lambda t: (t, 0, 0))] + _w_specs(),
        out_specs=pl.BlockSpec((1, B, HID), lambda t: (t, 0, 0)),
        out_shape=jax.ShapeDtypeStruct((T, B, HID), jnp.float32),
        scratch_shapes=[pltpu.VMEM((B, HID), jnp.float32)],
    )(emb_tbe, wih, whh, bih, bhh)


def _gru_backward_combine(emb_tbe, out_f, wih, whh, bih, bhh):
    rev = lambda t: (T - 1 - t, 0, 0)
    return pl.pallas_call(
        _bwd_body,
        grid=(T,),
        in_specs=[pl.BlockSpec((1, B, EMB), rev),
                  pl.BlockSpec((1, B, HID), rev)] + _w_specs(),
        out_specs=pl.BlockSpec((B, 8, 2 * HID),
                               lambda t: (0, (T - 1 - t) // 8, 0)),
        out_shape=jax.ShapeDtypeStruct((B, T, 2 * HID), jnp.float32),
        scratch_shapes=[pltpu.VMEM((B, HID), jnp.float32)],
    )(emb_tbe, out_f, wih, whh, bih, bhh)


def kernel(x, table, W_ih_f, W_hh_f, b_ih_f, b_hh_f,
           W_ih_b, W_hh_b, b_ih_b, b_hh_b):
    idx_2d = x.T.reshape(_NW * _NCHUNK, _CH).astype(jnp.int32)
    emb = _sc_gather(table, idx_2d).reshape(T, B, EMB)
    out_f = _gru_forward(emb, *_prep_w(W_ih_f, W_hh_f, b_ih_f, b_hh_f))
    return _gru_backward_combine(emb, out_f,
                                 *_prep_w(W_ih_b, W_hh_b, b_ih_b, b_hh_b))


# speedup vs baseline: 1.0597x; 1.0416x over previous
import functools

import jax
import jax.numpy as jnp
from jax import lax
from jax.experimental import pallas as pl
from jax.experimental.pallas import tpu as pltpu
from jax.experimental.pallas import tpu_sc as plsc

VOCAB = 1000000
EMB = 64
HID = 64
B = 4096
T = 200


_NC, _NS = 2, 16
_NW = _NC * _NS
_N_ROWS = B * T
_PER_W = _N_ROWS // _NW
_CH = 128
_NCHUNK = _PER_W // _CH


def _sc_gather(table, idx_2d):
    mesh = plsc.VectorSubcoreMesh(core_axis_name="c", subcore_axis_name="s")

    @functools.partial(
        pl.kernel,
        mesh=mesh,
        compiler_params=pltpu.CompilerParams(use_tc_tiling_on_sc=False),
        out_type=jax.ShapeDtypeStruct((_N_ROWS, EMB), jnp.float32),
        scratch_types=[
            pltpu.VMEM((_NCHUNK, _CH), jnp.int32),
            pltpu.VMEM((2, _CH, EMB), jnp.float32),
            pltpu.SemaphoreType.DMA,
            pltpu.SemaphoreType.DMA,
        ],
    )
    def k(table_hbm, idx_hbm, out_hbm, idx_v, rows_v, sem0, sem1):
        wid = lax.axis_index("s") * _NC + lax.axis_index("c")
        base = wid * _PER_W
        pltpu.sync_copy(idx_hbm.at[pl.ds(wid * _NCHUNK, _NCHUNK)], idx_v)
        sems = (sem0, sem1)

        def start(j, slot):
            pltpu.async_copy(table_hbm.at[idx_v.at[j]], rows_v.at[slot],
                             sems[slot])

        start(0, 0)
        start(1, 1)

        def pair(i, carry):
            for b in range(2):
                j = 2 * i + b
                pltpu.make_async_copy(table_hbm.at[idx_v.at[j]],
                                      rows_v.at[b], sems[b]).wait()
                pltpu.sync_copy(rows_v.at[b],
                                out_hbm.at[pl.ds(base + j * _CH, _CH)])

                @pl.when(j + 2 < _NCHUNK)
                def _start_next(b=b, j=j):
                    start(j + 2, b)
            return carry

        lax.fori_loop(0, _NCHUNK // 2, pair, 0)

    return k(table, idx_2d)


def _xpose_body(x_ref, out_ref):
    out_ref[...] = x_ref[...].T


def _transpose_x(x):
    return pl.pallas_call(
        _xpose_body,
        out_shape=jax.ShapeDtypeStruct((T, B), jnp.int32),
    )(x)


def _gru_math(x_t, h, wih, whh, bih, bhh):
    gi = lax.dot_general(x_t, wih, (((1,), (1,)), ((), ())),
                         preferred_element_type=jnp.float32) + bih
    gh = lax.dot_general(h, whh, (((1,), (1,)), ((), ())),
                         preferred_element_type=jnp.float32) + bhh
    r = jax.nn.sigmoid(gi[:, :HID] + gh[:, :HID])
    z = jax.nn.sigmoid(gi[:, HID:2 * HID] + gh[:, HID:2 * HID])
    n = jnp.tanh(gi[:, 2 * HID:] + r * gh[:, 2 * HID:])
    return (1.0 - z) * n + z * h


def _fwd_body(emb_ref, wih_ref, whh_ref, bih_ref, bhh_ref, out_ref, h_ref):
    t = pl.program_id(0)

    @pl.when(t == 0)
    def _():
        h_ref[...] = jnp.zeros((B, HID), jnp.float32)

    h_new = _gru_math(emb_ref[0], h_ref[...], wih_ref[...], whh_ref[...],
                      bih_ref[...], bhh_ref[...])
    h_ref[...] = h_new
    out_ref[0] = h_new


def _bwd_body(emb_ref, fwd_ref, wih_ref, whh_ref, bih_ref, bhh_ref,
              out_hbm, h_ref, cat_ref, sems):
    t = pl.program_id(0)
    tp = T - 1 - t
    slot = lax.rem(t, 2)

    @pl.when(t == 0)
    def _():
        h_ref[...] = jnp.zeros((B, HID), jnp.float32)

    @pl.when(t >= 2)
    def _():
        pltpu.make_async_copy(cat_ref.at[slot],
                              out_hbm.at[:, pl.ds(tp, 1), :],
                              sems.at[slot]).wait()

    h_new = _gru_math(emb_ref[0], h_ref[...], wih_ref[...], whh_ref[...],
                      bih_ref[...], bhh_ref[...])
    h_ref[...] = h_new
    cat_ref[slot, :, 0, :] = jnp.concatenate([fwd_ref[0], h_new], axis=-1)
    pltpu.make_async_copy(cat_ref.at[slot],
                          out_hbm.at[:, pl.ds(tp, 1), :],
                          sems.at[slot]).start()

    @pl.when(t == T - 1)
    def _():
        for s in range(2):
            pltpu.make_async_copy(cat_ref.at[s],
                                  out_hbm.at[:, pl.ds(tp, 1), :],
                                  sems.at[s]).wait()


def _w_specs():
    return [
        pl.BlockSpec((3 * HID, EMB), lambda t: (0, 0)),
        pl.BlockSpec((3 * HID, HID), lambda t: (0, 0)),
        pl.BlockSpec((1, 3 * HID), lambda t: (0, 0)),
        pl.BlockSpec((1, 3 * HID), lambda t: (0, 0)),
    ]


def _prep_w(W_ih, W_hh, b_ih, b_hh):
    return (W_ih, W_hh, b_ih[None, :], b_hh[None, :])


def _gru_forward(emb_tbe, wih, whh, bih, bhh):
    return pl.pallas_call(
        _fwd_body,
        grid=(T,),
        in_specs=[pl.BlockSpec((1, B, EMB), lambda t: (t, 0, 0))] + _w_specs(),
        out_specs=pl.BlockSpec((1, B, HID), lambda t: (t, 0, 0)),
        out_shape=jax.ShapeDtypeStruct((T, B, HID), jnp.float32),
        scratch_shapes=[pltpu.VMEM((B, HID), jnp.float32)],
    )(emb_tbe, wih, whh, bih, bhh)


def _gru_backward_combine(emb_tbe, out_f, wih, whh, bih, bhh):
    rev = lambda t: (T - 1 - t, 0, 0)
    return pl.pallas_call(
        _bwd_body,
        grid=(T,),
        in_specs=[pl.BlockSpec((1, B, EMB), rev),
                  pl.BlockSpec((1, B, HID), rev)] + _w_specs(),
        out_specs=pl.BlockSpec(memory_space=pl.ANY),
        out_shape=jax.ShapeDtypeStruct((B, T, 2 * HID), jnp.float32),
        scratch_shapes=[pltpu.VMEM((B, HID), jnp.float32),
                        pltpu.VMEM((2, B, 1, 2 * HID), jnp.float32),
                        pltpu.SemaphoreType.DMA((2,))],
    )(emb_tbe, out_f, wih, whh, bih, bhh)


def kernel(x, table, W_ih_f, W_hh_f, b_ih_f, b_hh_f,
           W_ih_b, W_hh_b, b_ih_b, b_hh_b):
    idx_2d = _transpose_x(x.astype(jnp.int32)).reshape(_NW * _NCHUNK, _CH)
    emb = _sc_gather(table, idx_2d).reshape(T, B, EMB)
    out_f = _gru_forward(emb, *_prep_w(W_ih_f, W_hh_f, b_ih_f, b_hh_f))
    return _gru_backward_combine(emb, out_f,
                                 *_prep_w(W_ih_b, W_hh_b, b_ih_b, b_hh_b))
